# baseline (speedup 1.0000x reference)
import functools

import jax
import jax.numpy as jnp
from jax import lax
from jax.experimental import pallas as pl
from jax.experimental.pallas import tpu as pltpu
from jax.experimental.pallas import tpu_sc as plsc

B, T = 1024, 50
ROW = 1000
ROWP = 1024
NUM_WORKERS = 32
NSPLIT = 2
NB = B // NSPLIT
PER_WORKER = NB // NUM_WORKERS
PAIRS = PER_WORKER // 2

_MESH = plsc.VectorSubcoreMesh(core_axis_name="c", subcore_axis_name="s")


@functools.partial(
    pl.kernel,
    mesh=_MESH,
    out_type=jax.ShapeDtypeStruct((NB, T, 8, 128), jnp.float32),
    scratch_types=[
        pltpu.VMEM((PER_WORKER, T), jnp.int32),
        pltpu.VMEM((T, 8, 128), jnp.float32),
        pltpu.VMEM((T, 8, 128), jnp.float32),
        pltpu.SemaphoreType.DMA,
        pltpu.SemaphoreType.DMA,
        pltpu.SemaphoreType.DMA,
        pltpu.SemaphoreType.DMA,
    ],
)
def _gather(idx_hbm, table_hbm, out_hbm, idx_v, b0, b1, sg0, sg1, ss0, ss1):
    wid = lax.axis_index("s") * 2 + lax.axis_index("c")
    base = wid * PER_WORKER
    pltpu.sync_copy(idx_hbm.at[pl.ds(base, PER_WORKER)], idx_v)

    def g_start(j, buf, sem):
        return pltpu.async_copy(table_hbm.at[idx_v.at[j]], buf, sem)

    def g_wait(j, buf, sem):
        pltpu.make_async_copy(table_hbm.at[idx_v.at[j]], buf, sem).wait()

    def s_start(j, buf, sem):
        return pltpu.async_copy(buf, out_hbm.at[base + j], sem)

    def s_wait(j, buf, sem):
        pltpu.make_async_copy(buf, out_hbm.at[base + j], sem).wait()

    d = g_start(0, b0, sg0)
    d.wait()
    d0 = s_start(0, b0, ss0)
    g_start(1, b1, sg1)
    d0.wait()
    g_start(2, b0, sg0)
    g_wait(1, b1, sg1)
    s_start(1, b1, ss1)

    def body(s, carry):
        j0 = 2 * s
        j1 = j0 + 1
        g_wait(j0, b0, sg0)
        dsc = s_start(j0, b0, ss0)
        s_wait(j1 - 2, b1, ss1)
        dg = g_start(j1, b1, sg1)
        dsc.wait()
        g_start(j0 + 2, b0, sg0)
        dg.wait()
        s_start(j1, b1, ss1)
        return carry

    lax.fori_loop(1, PAIRS - 1, body, 0)

    jA = PER_WORKER - 2
    jB = PER_WORKER - 1
    g_wait(jA, b0, sg0)
    dA = s_start(jA, b0, ss0)
    s_wait(jA - 1, b1, ss1)
    dB = g_start(jB, b1, sg1)
    dB.wait()
    dC = s_start(jB, b1, ss1)
    dA.wait()
    dC.wait()


def kernel(x, logits_table):
    table3 = jnp.pad(logits_table, ((0, 0), (0, ROWP - ROW))).reshape(
        1000, 8, 128
    )
    xi = x.astype(jnp.int32)
    parts = [
        _gather(xi[i * NB:(i + 1) * NB], table3).reshape(NB, T, ROWP)[
            :, :, :ROW
        ]
        for i in range(NSPLIT)
    ]
    return jnp.concatenate(parts, axis=0)


# speedup vs baseline: 1.2156x; 1.2156x over previous
import functools

import jax
import jax.numpy as jnp
from jax import lax
from jax.experimental import pallas as pl
from jax.experimental.pallas import tpu as pltpu
from jax.experimental.pallas import tpu_sc as plsc

B, T = 1024, 50
ROW = 1000
ROWP = 1024
NUM_WORKERS = 32
PER_WORKER = B // NUM_WORKERS
PAIRS = PER_WORKER // 2

_MESH = plsc.VectorSubcoreMesh(core_axis_name="c", subcore_axis_name="s")


@functools.partial(
    pl.kernel,
    mesh=_MESH,
    out_type=jax.ShapeDtypeStruct((B, T, 8, 128), jnp.float32),
    scratch_types=[
        pltpu.VMEM((PER_WORKER, T), jnp.int32),
        pltpu.VMEM((T, 8, 128), jnp.float32),
        pltpu.VMEM((T, 8, 128), jnp.float32),
        pltpu.SemaphoreType.DMA,
        pltpu.SemaphoreType.DMA,
        pltpu.SemaphoreType.DMA,
        pltpu.SemaphoreType.DMA,
    ],
)
def _gather(idx_hbm, table_hbm, out_hbm, idx_v, b0, b1, sg0, sg1, ss0, ss1):
    wid = lax.axis_index("s") * 2 + lax.axis_index("c")
    base = wid * PER_WORKER
    pltpu.sync_copy(idx_hbm.at[pl.ds(base, PER_WORKER)], idx_v)

    def g_start(j, buf, sem):
        return pltpu.async_copy(table_hbm.at[idx_v.at[j]], buf, sem)

    def g_wait(j, buf, sem):
        pltpu.make_async_copy(table_hbm.at[idx_v.at[j]], buf, sem).wait()

    def s_start(j, buf, sem):
        return pltpu.async_copy(buf, out_hbm.at[base + j], sem)

    def s_wait(j, buf, sem):
        pltpu.make_async_copy(buf, out_hbm.at[base + j], sem).wait()

    d = g_start(0, b0, sg0)
    d.wait()
    d0 = s_start(0, b0, ss0)
    g_start(1, b1, sg1)
    d0.wait()
    g_start(2, b0, sg0)
    g_wait(1, b1, sg1)
    s_start(1, b1, ss1)

    def body(s, carry):
        j0 = 2 * s
        j1 = j0 + 1
        g_wait(j0, b0, sg0)
        dsc = s_start(j0, b0, ss0)
        s_wait(j1 - 2, b1, ss1)
        dg = g_start(j1, b1, sg1)
        dsc.wait()
        g_start(j0 + 2, b0, sg0)
        dg.wait()
        s_start(j1, b1, ss1)
        return carry

    lax.fori_loop(1, PAIRS - 1, body, 0)

    jA = PER_WORKER - 2
    jB = PER_WORKER - 1
    g_wait(jA, b0, sg0)
    dA = s_start(jA, b0, ss0)
    s_wait(jA - 1, b1, ss1)
    dB = g_start(jB, b1, sg1)
    dB.wait()
    dC = s_start(jB, b1, ss1)
    dA.wait()
    dC.wait()


def kernel(x, logits_table):
    table3 = jnp.pad(logits_table, ((0, 0), (0, ROWP - ROW))).reshape(
        1000, 8, 128
    )
    out = _gather(x.astype(jnp.int32), table3)
    return out.reshape(B, T, ROWP)[:, :, :ROW]
